# baseline (speedup 1.0000x reference)
import functools

import jax
import jax.numpy as jnp
from jax import lax
from jax.experimental import pallas as pl
from jax.experimental.pallas import tpu as pltpu
from jax.experimental.pallas import tpu_sc as plsc

EPS = 1e-7
MIN_NORM = 1e-15

NS = 16
K = 64
L = 16


def _tc_logmap(x_ref, oa_ref, ob_ref):
    x = x_ref[...]
    col = lax.broadcasted_iota(jnp.int32, x.shape, 1)
    is_rest = col > 0
    xm = jnp.where(is_rest, x, 0.0)
    sq = jnp.sum(xm * xm, axis=1, keepdims=True)
    y_norm = jnp.maximum(jnp.sqrt(sq), MIN_NORM)
    theta = jnp.maximum(x[:, 0:1], 1.0 + EPS)
    z = jnp.sqrt(jnp.maximum(theta * theta - 1.0, EPS))
    arco = jnp.log(theta + z)
    res = jnp.where(is_rest, (arco / y_norm) * x, 0.0)
    h = x.shape[1] // 2
    oa_ref[...] = res[:, :h]
    ob_ref[...] = res[:, h:]


def _tc_post(o1a_ref, o1b_ref, o2a_ref, o2b_ref, h_ref):
    ta = o1a_ref[...] + o2a_ref[...]
    tb = o1b_ref[...] + o2b_ref[...]
    cola = lax.broadcasted_iota(jnp.int32, ta.shape, 1)
    is_rest = cola > 0
    tam = jnp.where(is_rest, ta, 0.0)
    x_sq = jnp.sum(tam * tam, axis=1, keepdims=True) + jnp.sum(
        tb * tb, axis=1, keepdims=True)
    x_norm = jnp.maximum(jnp.sqrt(x_sq), MIN_NORM)
    theta = x_norm
    e = jnp.exp(theta)
    einv = 1.0 / e
    cosh_t = 0.5 * (e + einv)
    coef = 0.5 * (e - einv) / x_norm
    ra = jnp.where(is_rest, coef * ta, 0.0)
    rb = coef * tb
    y_sq = jnp.sum(ra * ra, axis=1, keepdims=True) + jnp.sum(
        rb * rb, axis=1, keepdims=True)
    first = jnp.sqrt(jnp.maximum(1.0 + y_sq, EPS))
    del cosh_t
    ha = jnp.where(cola == 0, first, ra)
    h_ref[...] = jnp.concatenate([ha, rb], axis=1)


def _sc_spmm2(xab_ref, srcp_ref, dstp_ref, wp_ref, o1_ref, o2_ref,
              accum, gsem_a, gsem_b, *, n, c_chunks):
    h = xab_ref.shape[1]
    half = c_chunks // 2
    pl.run_scoped(
        functools.partial(_sc_spmm2_scoped, xab_ref, srcp_ref, dstp_ref,
                          wp_ref, o1_ref, o2_ref, accum, gsem_a, gsem_b,
                          n=n, c_chunks=c_chunks),
        pltpu.VMEM((half * K,), jnp.int32),
        pltpu.VMEM((half, K), jnp.int32),
        pltpu.VMEM((half * K,), jnp.float32),
        pltpu.VMEM((K, h), jnp.float32),
        pltpu.VMEM((K, h), jnp.float32),
    )


def _sc_spmm2_scoped(xab_ref, srcp_ref, dstp_ref, wp_ref, o1_ref, o2_ref,
                     accum, gsem_a, gsem_b, src_v, dst_v, w_v, buf_a, buf_b,
                     *, n, c_chunks):
    cid = lax.axis_index("c")
    sid = lax.axis_index("s")
    base = cid * n
    half = c_chunks // 2
    tpe = c_chunks * K
    stripe = (n // (NS * 8)) * 8
    last_rows = n - (NS - 1) * stripe
    n_full = stripe // K
    rem = stripe % K
    last_extra = last_rows - stripe

    def _for_stripe(emit):
        s0 = sid * stripe
        for j in range(n_full):
            emit(s0 + j * K, K)
        if rem:
            @pl.when(sid < NS - 1)
            def _():
                emit(s0 + n_full * K, rem)

        @pl.when(sid == NS - 1)
        def _():
            if rem + last_extra:
                emit(s0 + n_full * K, rem + last_extra)

    def _gidx(jj):
        return src_v.at[pl.ds(pl.multiple_of(jj * K, 8), K)]

    def _zero_accum():
        def _zrow(i, _):
            for t in range(8):
                buf_a[i, pl.ds(t * L, L)] = jnp.zeros((L,), jnp.float32)
            return 0
        lax.fori_loop(0, K, _zrow, 0)

        def _z(off, nrows):
            pltpu.sync_copy(buf_a.at[pl.ds(0, nrows)],
                            accum.at[pl.ds(off, nrows)])
        _for_stripe(_z)

    def _weight_chunk(buf, jj):
        @plsc.parallel_loop(0, K // L, unroll=4)
        def _grp(g):
            w16 = w_v[pl.ds(jj * K + g * L, L)]
            for lane in range(L):
                ei = g * L + lane
                w = w16[lane]
                for t in range(8):
                    sl = pl.ds(t * L, L)
                    buf[ei, sl] = buf[ei, sl] * w

    def _layer(table_ref, out_ref):
        _zero_accum()
        plsc.subcore_barrier()

        def _half_body(hf, _):
            off_e = hf * half * K
            pltpu.sync_copy(
                srcp_ref.at[pl.ds(cid * NS * tpe + sid * tpe + off_e,
                                  half * K)], src_v)
            pltpu.sync_copy(wp_ref.at[pl.ds(sid * tpe + off_e, half * K)],
                            w_v)
            pltpu.sync_copy(dstp_ref.at[sid, pl.ds(hf * half, half)], dst_v)

            pltpu.async_copy(table_ref.at[_gidx(0)], buf_a, gsem_a)
            pltpu.async_copy(table_ref.at[_gidx(1)], buf_b, gsem_b)

            def _pair(i, _):
                j = 2 * i
                for (off, buf, gsem) in ((0, buf_a, gsem_a),
                                         (1, buf_b, gsem_b)):
                    jj = j + off
                    pltpu.make_async_copy(
                        table_ref.at[_gidx(jj)], buf, gsem).wait()
                    _weight_chunk(buf, jj)
                    pltpu.sync_copy(buf, accum.at[dst_v.at[jj]], add=True)
                    nxt = jj + 2

                    @pl.when(nxt < half)
                    def _():
                        pltpu.async_copy(table_ref.at[_gidx(nxt)], buf, gsem)
                return 0
            lax.fori_loop(0, half // 2, _pair, 0)
            return 0
        lax.fori_loop(0, 2, _half_body, 0)
        plsc.subcore_barrier()

        def _w(off, nrows):
            pltpu.sync_copy(accum.at[pl.ds(off, nrows)],
                            out_ref.at[pl.ds(base + off, nrows)])
        _for_stripe(_w)
        plsc.subcore_barrier()

    _layer(xab_ref, o1_ref)
    _layer(o1_ref, o2_ref)


def kernel(x, edge_index, edge_weight):
    n, d = x.shape
    e = edge_weight.shape[0]
    h = d // 2
    rb = 1000
    nb = n // rb

    per_tile = -(-e // NS)
    c_chunks = ((-(-per_tile // K) + 3) // 4) * 4
    tpe = c_chunks * K
    total = NS * tpe

    src = edge_index[1].astype(jnp.int32)
    dst = edge_index[0].astype(jnp.int32)
    w = edge_weight.astype(jnp.float32)
    pad = total - e
    src_pad = jnp.pad(src, (0, pad))
    srcp = jnp.concatenate([src_pad, src_pad + n])
    dstp = jnp.pad(dst, (0, pad)).reshape(NS, c_chunks, K)
    wp = jnp.pad(w, (0, pad))

    xa, xb = pl.pallas_call(
        _tc_logmap,
        grid=(nb,),
        in_specs=[pl.BlockSpec((rb, d), lambda i: (i, 0))],
        out_specs=[pl.BlockSpec((rb, h), lambda i: (i, 0))] * 2,
        out_shape=[jax.ShapeDtypeStruct((n, h), jnp.float32)] * 2,
    )(x)
    xab = jnp.concatenate([xa, xb], axis=0)

    mesh = plsc.VectorSubcoreMesh(core_axis_name="c", subcore_axis_name="s")
    o1, o2 = pl.kernel(
        functools.partial(_sc_spmm2, n=n, c_chunks=c_chunks),
        out_type=(jax.ShapeDtypeStruct((2 * n, h), jnp.float32),
                  jax.ShapeDtypeStruct((2 * n, h), jnp.float32)),
        mesh=mesh,
        scratch_types=[
            pltpu.VMEM_SHARED((n, h), jnp.float32),
            pltpu.SemaphoreType.DMA,
            pltpu.SemaphoreType.DMA,
        ],
    )(xab, srcp, dstp, wp)

    return pl.pallas_call(
        _tc_post,
        grid=(nb,),
        in_specs=[
            pl.BlockSpec((rb, h), lambda i: (i, 0)),
            pl.BlockSpec((rb, h), lambda i: (i + nb, 0)),
            pl.BlockSpec((rb, h), lambda i: (i, 0)),
            pl.BlockSpec((rb, h), lambda i: (i + nb, 0)),
        ],
        out_specs=pl.BlockSpec((rb, d), lambda i: (i, 0)),
        out_shape=jax.ShapeDtypeStruct((n, d), jnp.float32),
    )(o1, o1, o2, o2)


# speedup vs baseline: 1.0058x; 1.0058x over previous
import functools

import jax
import jax.numpy as jnp
from jax import lax
from jax.experimental import pallas as pl
from jax.experimental.pallas import tpu as pltpu
from jax.experimental.pallas import tpu_sc as plsc

EPS = 1e-7
MIN_NORM = 1e-15

NS = 16
K = 64
L = 16


def _tc_logmap(x_ref, oa_ref, ob_ref):
    x = x_ref[...]
    col = lax.broadcasted_iota(jnp.int32, x.shape, 1)
    is_rest = col > 0
    xm = jnp.where(is_rest, x, 0.0)
    sq = jnp.sum(xm * xm, axis=1, keepdims=True)
    y_norm = jnp.maximum(jnp.sqrt(sq), MIN_NORM)
    theta = jnp.maximum(x[:, 0:1], 1.0 + EPS)
    z = jnp.sqrt(jnp.maximum(theta * theta - 1.0, EPS))
    arco = jnp.log(theta + z)
    res = jnp.where(is_rest, (arco / y_norm) * x, 0.0)
    h = x.shape[1] // 2
    oa_ref[...] = res[:, :h]
    ob_ref[...] = res[:, h:]


def _tc_post(o1a_ref, o1b_ref, o2a_ref, o2b_ref, h_ref):
    ta = o1a_ref[...] + o2a_ref[...]
    tb = o1b_ref[...] + o2b_ref[...]
    cola = lax.broadcasted_iota(jnp.int32, ta.shape, 1)
    is_rest = cola > 0
    tam = jnp.where(is_rest, ta, 0.0)
    x_sq = jnp.sum(tam * tam, axis=1, keepdims=True) + jnp.sum(
        tb * tb, axis=1, keepdims=True)
    x_norm = jnp.maximum(jnp.sqrt(x_sq), MIN_NORM)
    theta = x_norm
    e = jnp.exp(theta)
    einv = 1.0 / e
    cosh_t = 0.5 * (e + einv)
    coef = 0.5 * (e - einv) / x_norm
    ra = jnp.where(is_rest, coef * ta, 0.0)
    rb = coef * tb
    y_sq = jnp.sum(ra * ra, axis=1, keepdims=True) + jnp.sum(
        rb * rb, axis=1, keepdims=True)
    first = jnp.sqrt(jnp.maximum(1.0 + y_sq, EPS))
    del cosh_t
    ha = jnp.where(cola == 0, first, ra)
    h_ref[...] = jnp.concatenate([ha, rb], axis=1)


def _sc_spmm2(xab_ref, srcp_ref, dstp_ref, wp_ref, o1_ref, o2_ref,
              accum, gsem_a, gsem_b, *, n, c_chunks):
    h = xab_ref.shape[1]
    half = c_chunks // 2
    pl.run_scoped(
        functools.partial(_sc_spmm2_scoped, xab_ref, srcp_ref, dstp_ref,
                          wp_ref, o1_ref, o2_ref, accum, gsem_a, gsem_b,
                          n=n, c_chunks=c_chunks),
        pltpu.VMEM((half * K,), jnp.int32),
        pltpu.VMEM((half, K), jnp.int32),
        pltpu.VMEM((half * K,), jnp.float32),
        pltpu.VMEM((K, h), jnp.float32),
        pltpu.VMEM((K, h), jnp.float32),
    )


def _sc_spmm2_scoped(xab_ref, srcp_ref, dstp_ref, wp_ref, o1_ref, o2_ref,
                     accum, gsem_a, gsem_b, src_v, dst_v, w_v, buf_a, buf_b,
                     *, n, c_chunks):
    cid = lax.axis_index("c")
    sid = lax.axis_index("s")
    base = cid * n
    half = c_chunks // 2
    tpe = c_chunks * K
    stripe = (n // (NS * 8)) * 8
    last_rows = n - (NS - 1) * stripe
    n_full = stripe // K
    rem = stripe % K
    last_extra = last_rows - stripe

    def _for_stripe(emit):
        s0 = sid * stripe
        for j in range(n_full):
            emit(s0 + j * K, K)
        if rem:
            @pl.when(sid < NS - 1)
            def _():
                emit(s0 + n_full * K, rem)

        @pl.when(sid == NS - 1)
        def _():
            if rem + last_extra:
                emit(s0 + n_full * K, rem + last_extra)

    ksub = K // 2

    def _gsub(jj, u):
        return src_v.at[pl.ds(pl.multiple_of(jj * K + u * ksub, 8), ksub)]

    def _fire(table_ref, jj, buf, gsem):
        for u in range(2):
            pltpu.async_copy(table_ref.at[_gsub(jj, u)],
                             buf.at[pl.ds(u * ksub, ksub)], gsem)

    def _drain(table_ref, jj, buf, gsem):
        for u in range(2):
            pltpu.make_async_copy(table_ref.at[_gsub(jj, u)],
                                  buf.at[pl.ds(u * ksub, ksub)], gsem).wait()

    def _zero_accum():
        def _zrow(i, _):
            for t in range(8):
                buf_a[i, pl.ds(t * L, L)] = jnp.zeros((L,), jnp.float32)
            return 0
        lax.fori_loop(0, K, _zrow, 0)

        def _z(off, nrows):
            pltpu.sync_copy(buf_a.at[pl.ds(0, nrows)],
                            accum.at[pl.ds(off, nrows)])
        _for_stripe(_z)

    def _weight_chunk(buf, jj):
        @plsc.parallel_loop(0, K // L, unroll=2)
        def _grp(g):
            w16 = w_v[pl.ds(jj * K + g * L, L)]
            for lane in range(L):
                ei = g * L + lane
                w = w16[lane]
                for t in range(8):
                    sl = pl.ds(t * L, L)
                    buf[ei, sl] = buf[ei, sl] * w

    def _layer(table_ref, out_ref):
        _zero_accum()
        plsc.subcore_barrier()

        def _half_body(hf, _):
            off_e = hf * half * K
            pltpu.sync_copy(
                srcp_ref.at[pl.ds(cid * NS * tpe + sid * tpe + off_e,
                                  half * K)], src_v)
            pltpu.sync_copy(wp_ref.at[pl.ds(sid * tpe + off_e, half * K)],
                            w_v)
            pltpu.sync_copy(dstp_ref.at[sid, pl.ds(hf * half, half)], dst_v)

            _fire(table_ref, 0, buf_a, gsem_a)
            _fire(table_ref, 1, buf_b, gsem_b)

            def _pair(i, _):
                j = 2 * i
                for (off, buf, gsem) in ((0, buf_a, gsem_a),
                                         (1, buf_b, gsem_b)):
                    jj = j + off
                    _drain(table_ref, jj, buf, gsem)
                    _weight_chunk(buf, jj)
                    pltpu.sync_copy(buf, accum.at[dst_v.at[jj]], add=True)
                    nxt = jj + 2

                    @pl.when(nxt < half)
                    def _():
                        _fire(table_ref, nxt, buf, gsem)
                return 0
            lax.fori_loop(0, half // 2, _pair, 0)
            return 0
        lax.fori_loop(0, 2, _half_body, 0)
        plsc.subcore_barrier()

        def _w(off, nrows):
            pltpu.sync_copy(accum.at[pl.ds(off, nrows)],
                            out_ref.at[pl.ds(base + off, nrows)])
        _for_stripe(_w)
        plsc.subcore_barrier()

    _layer(xab_ref, o1_ref)
    _layer(o1_ref, o2_ref)


def kernel(x, edge_index, edge_weight):
    n, d = x.shape
    e = edge_weight.shape[0]
    h = d // 2
    rb = 1000
    nb = n // rb

    per_tile = -(-e // NS)
    c_chunks = ((-(-per_tile // K) + 3) // 4) * 4
    tpe = c_chunks * K
    total = NS * tpe

    src = edge_index[1].astype(jnp.int32)
    dst = edge_index[0].astype(jnp.int32)
    w = edge_weight.astype(jnp.float32)
    pad = total - e
    src_pad = jnp.pad(src, (0, pad))
    srcp = jnp.concatenate([src_pad, src_pad + n])
    dstp = jnp.pad(dst, (0, pad)).reshape(NS, c_chunks, K)
    wp = jnp.pad(w, (0, pad))

    xa, xb = pl.pallas_call(
        _tc_logmap,
        grid=(nb,),
        in_specs=[pl.BlockSpec((rb, d), lambda i: (i, 0))],
        out_specs=[pl.BlockSpec((rb, h), lambda i: (i, 0))] * 2,
        out_shape=[jax.ShapeDtypeStruct((n, h), jnp.float32)] * 2,
    )(x)
    xab = jnp.concatenate([xa, xb], axis=0)

    mesh = plsc.VectorSubcoreMesh(core_axis_name="c", subcore_axis_name="s")
    o1, o2 = pl.kernel(
        functools.partial(_sc_spmm2, n=n, c_chunks=c_chunks),
        out_type=(jax.ShapeDtypeStruct((2 * n, h), jnp.float32),
                  jax.ShapeDtypeStruct((2 * n, h), jnp.float32)),
        mesh=mesh,
        scratch_types=[
            pltpu.VMEM_SHARED((n, h), jnp.float32),
            pltpu.SemaphoreType.DMA,
            pltpu.SemaphoreType.DMA,
        ],
    )(xab, srcp, dstp, wp)

    return pl.pallas_call(
        _tc_post,
        grid=(nb,),
        in_specs=[
            pl.BlockSpec((rb, h), lambda i: (i, 0)),
            pl.BlockSpec((rb, h), lambda i: (i + nb, 0)),
            pl.BlockSpec((rb, h), lambda i: (i, 0)),
            pl.BlockSpec((rb, h), lambda i: (i + nb, 0)),
        ],
        out_specs=pl.BlockSpec((rb, d), lambda i: (i, 0)),
        out_shape=jax.ShapeDtypeStruct((n, d), jnp.float32),
    )(o1, o1, o2, o2)


# speedup vs baseline: 1.0118x; 1.0059x over previous
import functools

import jax
import jax.numpy as jnp
from jax import lax
from jax.experimental import pallas as pl
from jax.experimental.pallas import tpu as pltpu
from jax.experimental.pallas import tpu_sc as plsc

EPS = 1e-7
MIN_NORM = 1e-15

NS = 16
K = 64
L = 16


def _tc_logmap(x_ref, oa_ref, ob_ref):
    x = x_ref[...]
    col = lax.broadcasted_iota(jnp.int32, x.shape, 1)
    is_rest = col > 0
    xm = jnp.where(is_rest, x, 0.0)
    sq = jnp.sum(xm * xm, axis=1, keepdims=True)
    y_norm = jnp.maximum(jnp.sqrt(sq), MIN_NORM)
    theta = jnp.maximum(x[:, 0:1], 1.0 + EPS)
    z = jnp.sqrt(jnp.maximum(theta * theta - 1.0, EPS))
    arco = jnp.log(theta + z)
    res = jnp.where(is_rest, (arco / y_norm) * x, 0.0)
    h = x.shape[1] // 2
    oa_ref[...] = res[:, :h]
    ob_ref[...] = res[:, h:]


def _tc_post(o1a_ref, o1b_ref, o2a_ref, o2b_ref, h_ref):
    ta = o1a_ref[...] + o2a_ref[...]
    tb = o1b_ref[...] + o2b_ref[...]
    cola = lax.broadcasted_iota(jnp.int32, ta.shape, 1)
    is_rest = cola > 0
    tam = jnp.where(is_rest, ta, 0.0)
    x_sq = jnp.sum(tam * tam, axis=1, keepdims=True) + jnp.sum(
        tb * tb, axis=1, keepdims=True)
    x_norm = jnp.maximum(jnp.sqrt(x_sq), MIN_NORM)
    theta = x_norm
    e = jnp.exp(theta)
    einv = 1.0 / e
    cosh_t = 0.5 * (e + einv)
    coef = 0.5 * (e - einv) / x_norm
    ra = jnp.where(is_rest, coef * ta, 0.0)
    rb = coef * tb
    y_sq = jnp.sum(ra * ra, axis=1, keepdims=True) + jnp.sum(
        rb * rb, axis=1, keepdims=True)
    first = jnp.sqrt(jnp.maximum(1.0 + y_sq, EPS))
    del cosh_t
    ha = jnp.where(cola == 0, first, ra)
    h_ref[...] = jnp.concatenate([ha, rb], axis=1)


def _sc_spmm2(xab_ref, srcp_ref, dstp_ref, wp_ref, o1_ref, o2_ref,
              accum, gsem_a, gsem_b, zsem, *, n, c_chunks):
    h = xab_ref.shape[1]
    half = c_chunks // 2
    pl.run_scoped(
        functools.partial(_sc_spmm2_scoped, xab_ref, srcp_ref, dstp_ref,
                          wp_ref, o1_ref, o2_ref, accum, gsem_a, gsem_b,
                          zsem, n=n, c_chunks=c_chunks),
        pltpu.VMEM((half * K,), jnp.int32),
        pltpu.VMEM((half, K), jnp.int32),
        pltpu.VMEM((half * K,), jnp.float32),
        pltpu.VMEM((K, h), jnp.float32),
        pltpu.VMEM((K, h), jnp.float32),
    )


def _sc_spmm2_scoped(xab_ref, srcp_ref, dstp_ref, wp_ref, o1_ref, o2_ref,
                     accum, gsem_a, gsem_b, zsem, src_v, dst_v, w_v,
                     buf_a, buf_b, *, n, c_chunks):
    cid = lax.axis_index("c")
    sid = lax.axis_index("s")
    base = cid * n
    half = c_chunks // 2
    tpe = c_chunks * K
    stripe = (n // (NS * 8)) * 8
    last_rows = n - (NS - 1) * stripe
    n_full = stripe // K
    rem = stripe % K
    last_extra = last_rows - stripe

    def _for_stripe(emit):
        s0 = sid * stripe
        for j in range(n_full):
            emit(s0 + j * K, K)
        if rem:
            @pl.when(sid < NS - 1)
            def _():
                emit(s0 + n_full * K, rem)

        @pl.when(sid == NS - 1)
        def _():
            if rem + last_extra:
                emit(s0 + n_full * K, rem + last_extra)

    ksub = K // 2

    def _gsub(jj, u):
        return src_v.at[pl.ds(pl.multiple_of(jj * K + u * ksub, 8), ksub)]

    def _fire(table_ref, jj, buf, gsem):
        for u in range(2):
            pltpu.async_copy(table_ref.at[_gsub(jj, u)],
                             buf.at[pl.ds(u * ksub, ksub)], gsem)

    def _drain(table_ref, jj, buf, gsem):
        for u in range(2):
            pltpu.make_async_copy(table_ref.at[_gsub(jj, u)],
                                  buf.at[pl.ds(u * ksub, ksub)], gsem).wait()

    def _zero_accum():
        def _zrow(i, _):
            for t in range(8):
                buf_a[i, pl.ds(t * L, L)] = jnp.zeros((L,), jnp.float32)
            return 0
        lax.fori_loop(0, K, _zrow, 0)

        def _z(off, nrows):
            pltpu.async_copy(buf_a.at[pl.ds(0, nrows)],
                             accum.at[pl.ds(off, nrows)], zsem)
        _for_stripe(_z)

        def _zw(off, nrows):
            pltpu.make_async_copy(buf_a.at[pl.ds(0, nrows)],
                                  accum.at[pl.ds(off, nrows)], zsem).wait()
        _for_stripe(_zw)

    def _weight_chunk(buf, jj):
        @plsc.parallel_loop(0, K // L, unroll=2)
        def _grp(g):
            w16 = w_v[pl.ds(jj * K + g * L, L)]
            for lane in range(L):
                ei = g * L + lane
                w = w16[lane]
                for t in range(8):
                    sl = pl.ds(t * L, L)
                    buf[ei, sl] = buf[ei, sl] * w

    def _layer(table_ref, out_ref):
        _zero_accum()
        plsc.subcore_barrier()

        def _half_body(hf, _):
            off_e = hf * half * K
            pltpu.sync_copy(
                srcp_ref.at[pl.ds(cid * NS * tpe + sid * tpe + off_e,
                                  half * K)], src_v)
            pltpu.sync_copy(wp_ref.at[pl.ds(sid * tpe + off_e, half * K)],
                            w_v)
            pltpu.sync_copy(dstp_ref.at[sid, pl.ds(hf * half, half)], dst_v)

            _fire(table_ref, 0, buf_a, gsem_a)
            _fire(table_ref, 1, buf_b, gsem_b)

            def _pair(i, _):
                j = 2 * i
                for (off, buf, gsem) in ((0, buf_a, gsem_a),
                                         (1, buf_b, gsem_b)):
                    jj = j + off
                    _drain(table_ref, jj, buf, gsem)
                    _weight_chunk(buf, jj)
                    pltpu.sync_copy(buf, accum.at[dst_v.at[jj]], add=True)
                    nxt = jj + 2

                    @pl.when(nxt < half)
                    def _():
                        _fire(table_ref, nxt, buf, gsem)
                return 0
            lax.fori_loop(0, half // 2, _pair, 0)
            return 0
        lax.fori_loop(0, 2, _half_body, 0)
        plsc.subcore_barrier()

        def _w(off, nrows):
            pltpu.async_copy(accum.at[pl.ds(off, nrows)],
                             out_ref.at[pl.ds(base + off, nrows)], zsem)
        _for_stripe(_w)

        def _ww(off, nrows):
            pltpu.make_async_copy(accum.at[pl.ds(off, nrows)],
                                  out_ref.at[pl.ds(base + off, nrows)],
                                  zsem).wait()
        _for_stripe(_ww)
        plsc.subcore_barrier()

    _layer(xab_ref, o1_ref)
    _layer(o1_ref, o2_ref)


def kernel(x, edge_index, edge_weight):
    n, d = x.shape
    e = edge_weight.shape[0]
    h = d // 2
    rb = 1000
    nb = n // rb

    per_tile = -(-e // NS)
    c_chunks = ((-(-per_tile // K) + 3) // 4) * 4
    tpe = c_chunks * K
    total = NS * tpe

    src = edge_index[1].astype(jnp.int32)
    dst = edge_index[0].astype(jnp.int32)
    w = edge_weight.astype(jnp.float32)
    pad = total - e
    src_pad = jnp.pad(src, (0, pad))
    srcp = jnp.concatenate([src_pad, src_pad + n])
    dstp = jnp.pad(dst, (0, pad)).reshape(NS, c_chunks, K)
    wp = jnp.pad(w, (0, pad))

    xa, xb = pl.pallas_call(
        _tc_logmap,
        grid=(nb,),
        in_specs=[pl.BlockSpec((rb, d), lambda i: (i, 0))],
        out_specs=[pl.BlockSpec((rb, h), lambda i: (i, 0))] * 2,
        out_shape=[jax.ShapeDtypeStruct((n, h), jnp.float32)] * 2,
    )(x)
    xab = jnp.concatenate([xa, xb], axis=0)

    mesh = plsc.VectorSubcoreMesh(core_axis_name="c", subcore_axis_name="s")
    o1, o2 = pl.kernel(
        functools.partial(_sc_spmm2, n=n, c_chunks=c_chunks),
        out_type=(jax.ShapeDtypeStruct((2 * n, h), jnp.float32),
                  jax.ShapeDtypeStruct((2 * n, h), jnp.float32)),
        mesh=mesh,
        scratch_types=[
            pltpu.VMEM_SHARED((n, h), jnp.float32),
            pltpu.SemaphoreType.DMA,
            pltpu.SemaphoreType.DMA,
            pltpu.SemaphoreType.DMA,
        ],
    )(xab, srcp, dstp, wp)

    return pl.pallas_call(
        _tc_post,
        grid=(nb,),
        in_specs=[
            pl.BlockSpec((rb, h), lambda i: (i, 0)),
            pl.BlockSpec((rb, h), lambda i: (i + nb, 0)),
            pl.BlockSpec((rb, h), lambda i: (i, 0)),
            pl.BlockSpec((rb, h), lambda i: (i + nb, 0)),
        ],
        out_specs=pl.BlockSpec((rb, d), lambda i: (i, 0)),
        out_shape=jax.ShapeDtypeStruct((n, d), jnp.float32),
    )(o1, o1, o2, o2)
